# baseline (speedup 1.0000x reference)
import jax
import jax.numpy as jnp
from jax.experimental import pallas as pl
from jax.experimental.pallas import tpu as pltpu

_N = 16384
_D = 768
_TM = 1024
_STEPS = _N // _TM
_NBUF = 5


def _pipelined_kernel(x_hbm, w_ref, b_ref, o_hbm, x_buf, o_buf, in_sems, out_sems):
    def in_copy(i):
        slot = i % _NBUF
        return pltpu.make_async_copy(
            x_hbm.at[pl.ds(i * _TM, _TM), :], x_buf.at[slot], in_sems.at[slot])

    def out_copy(i):
        slot = i % _NBUF
        return pltpu.make_async_copy(
            o_buf.at[slot], o_hbm.at[pl.ds(i * _TM, _TM), :], out_sems.at[slot])

    for i in range(_NBUF):
        in_copy(i).start()

    wb = w_ref[...].astype(jnp.bfloat16)
    bias = b_ref[...]
    for i in range(_STEPS):
        slot = i % _NBUF
        in_copy(i).wait()
        if i >= _NBUF:
            out_copy(i - _NBUF).wait()
        y = jax.lax.dot_general(
            x_buf[slot].astype(jnp.bfloat16), wb,
            dimension_numbers=(((1,), (1,)), ((), ())),
            preferred_element_type=jnp.float32)
        o_buf[slot] = y + bias
        out_copy(i).start()
        if i + _NBUF < _STEPS:
            in_copy(i + _NBUF).start()
    for i in range(_STEPS - _NBUF, _STEPS):
        out_copy(i).wait()


@jax.jit
def kernel(hidden_states, W0, b0, W1, b1):
    B, S, D = hidden_states.shape
    x = hidden_states.reshape(B * S, D)

    out = pl.pallas_call(
        _pipelined_kernel,
        in_specs=[
            pl.BlockSpec(memory_space=pl.ANY),
            pl.BlockSpec(memory_space=pltpu.MemorySpace.VMEM),
            pl.BlockSpec(memory_space=pltpu.MemorySpace.VMEM),
        ],
        out_specs=pl.BlockSpec(memory_space=pl.ANY),
        out_shape=jax.ShapeDtypeStruct((B * S, D), jnp.float32),
        scratch_shapes=[
            pltpu.VMEM((_NBUF, _TM, _D), jnp.float32),
            pltpu.VMEM((_NBUF, _TM, _D), jnp.float32),
            pltpu.SemaphoreType.DMA((_NBUF,)),
            pltpu.SemaphoreType.DMA((_NBUF,)),
        ],
    )(x, W0, b0.reshape(1, D))
    return out.reshape(B, S, D)


# speedup vs baseline: 1.0425x; 1.0425x over previous
import jax
import jax.numpy as jnp
from jax.experimental import pallas as pl
from jax.experimental.pallas import tpu as pltpu

_N = 16384
_D = 768
_NBUF = 5
_SIZES = [256, 256, 512] + [1024] * 14 + [512, 256, 256]
_OFFS = [sum(_SIZES[:i]) for i in range(len(_SIZES))]
_STEPS = len(_SIZES)
_TMAX = max(_SIZES)


def _pipelined_kernel(x_hbm, w_ref, b_ref, o_hbm, x_buf, o_buf, in_sems, out_sems):
    def in_copy(i):
        slot = i % _NBUF
        return pltpu.make_async_copy(
            x_hbm.at[pl.ds(_OFFS[i], _SIZES[i]), :],
            x_buf.at[slot, pl.ds(0, _SIZES[i]), :],
            in_sems.at[slot])

    def out_copy(i):
        slot = i % _NBUF
        return pltpu.make_async_copy(
            o_buf.at[slot, pl.ds(0, _SIZES[i]), :],
            o_hbm.at[pl.ds(_OFFS[i], _SIZES[i]), :],
            out_sems.at[slot])

    for i in range(_NBUF):
        in_copy(i).start()

    for i in range(_STEPS):
        slot = i % _NBUF
        in_copy(i).wait()
        if i >= _NBUF:
            out_copy(i - _NBUF).wait()
        y = jax.lax.dot_general(
            x_buf[slot, pl.ds(0, _SIZES[i]), :].astype(jnp.bfloat16),
            w_ref[...].astype(jnp.bfloat16),
            dimension_numbers=(((1,), (1,)), ((), ())),
            preferred_element_type=jnp.float32)
        o_buf[slot, pl.ds(0, _SIZES[i]), :] = y + b_ref[...]
        out_copy(i).start()
        if i + _NBUF < _STEPS:
            in_copy(i + _NBUF).start()
    for i in range(_STEPS - _NBUF, _STEPS):
        out_copy(i).wait()


@jax.jit
def kernel(hidden_states, W0, b0, W1, b1):
    B, S, D = hidden_states.shape
    x = hidden_states.reshape(B * S, D)

    out = pl.pallas_call(
        _pipelined_kernel,
        in_specs=[
            pl.BlockSpec(memory_space=pl.ANY),
            pl.BlockSpec(memory_space=pltpu.MemorySpace.VMEM),
            pl.BlockSpec(memory_space=pltpu.MemorySpace.VMEM),
        ],
        out_specs=pl.BlockSpec(memory_space=pl.ANY),
        out_shape=jax.ShapeDtypeStruct((B * S, D), jnp.float32),
        scratch_shapes=[
            pltpu.VMEM((_NBUF, _TMAX, _D), jnp.float32),
            pltpu.VMEM((_NBUF, _TMAX, _D), jnp.float32),
            pltpu.SemaphoreType.DMA((_NBUF,)),
            pltpu.SemaphoreType.DMA((_NBUF,)),
        ],
    )(x, W0, b0.reshape(1, D))
    return out.reshape(B, S, D)


# speedup vs baseline: 1.0441x; 1.0015x over previous
import jax
import jax.numpy as jnp
from jax.experimental import pallas as pl
from jax.experimental.pallas import tpu as pltpu

_N = 16384
_D = 768
_NBUF = 6
_SIZES = [256, 256, 512] + [1024] * 14 + [512, 256, 256]
_OFFS = [sum(_SIZES[:i]) for i in range(len(_SIZES))]
_STEPS = len(_SIZES)
_TMAX = max(_SIZES)


def _pipelined_kernel(x_hbm, w_ref, b_ref, o_hbm, x_buf, o_buf, in_sems, out_sems):
    def in_copy(i):
        slot = i % _NBUF
        return pltpu.make_async_copy(
            x_hbm.at[pl.ds(_OFFS[i], _SIZES[i]), :],
            x_buf.at[slot, pl.ds(0, _SIZES[i]), :],
            in_sems.at[slot])

    def out_copy(i):
        slot = i % _NBUF
        return pltpu.make_async_copy(
            o_buf.at[slot, pl.ds(0, _SIZES[i]), :],
            o_hbm.at[pl.ds(_OFFS[i], _SIZES[i]), :],
            out_sems.at[slot])

    for i in range(_NBUF):
        in_copy(i).start()

    for i in range(_STEPS):
        slot = i % _NBUF
        in_copy(i).wait()
        if i >= _NBUF:
            out_copy(i - _NBUF).wait()
        y = jax.lax.dot_general(
            x_buf[slot, pl.ds(0, _SIZES[i]), :].astype(jnp.bfloat16),
            w_ref[...].astype(jnp.bfloat16),
            dimension_numbers=(((1,), (1,)), ((), ())),
            preferred_element_type=jnp.float32)
        o_buf[slot, pl.ds(0, _SIZES[i]), :] = y + b_ref[...]
        out_copy(i).start()
        if i + _NBUF < _STEPS:
            in_copy(i + _NBUF).start()
    for i in range(_STEPS - _NBUF, _STEPS):
        out_copy(i).wait()


@jax.jit
def kernel(hidden_states, W0, b0, W1, b1):
    B, S, D = hidden_states.shape
    x = hidden_states.reshape(B * S, D)

    out = pl.pallas_call(
        _pipelined_kernel,
        in_specs=[
            pl.BlockSpec(memory_space=pl.ANY),
            pl.BlockSpec(memory_space=pltpu.MemorySpace.VMEM),
            pl.BlockSpec(memory_space=pltpu.MemorySpace.VMEM),
        ],
        out_specs=pl.BlockSpec(memory_space=pl.ANY),
        out_shape=jax.ShapeDtypeStruct((B * S, D), jnp.float32),
        scratch_shapes=[
            pltpu.VMEM((_NBUF, _TMAX, _D), jnp.float32),
            pltpu.VMEM((_NBUF, _TMAX, _D), jnp.float32),
            pltpu.SemaphoreType.DMA((_NBUF,)),
            pltpu.SemaphoreType.DMA((_NBUF,)),
        ],
    )(x, W0, b0.reshape(1, D))
    return out.reshape(B, S, D)
